# baseline (speedup 1.0000x reference)
import functools

import jax
import jax.numpy as jnp
from jax import lax
from jax.experimental import pallas as pl
from jax.experimental.pallas import tpu as pltpu
from jax.experimental.pallas import tpu_sc as plsc

VOCAB = 1000
HID = 128
B = 4096
L = 50

NC = 2
NS = 16
NW = NC * NS
ROWS = 2 * B
R_PER_W = ROWS // NW
CHUNK_ROWS = 16
N_CHUNKS = R_PER_W // CHUNK_ROWS
CHUNK_IDX = CHUNK_ROWS * L
HIDW = HID // 2
NQ = HIDW // 16

_mesh = plsc.VectorSubcoreMesh(core_axis_name="c", subcore_axis_name="s")


@functools.partial(
    pl.kernel,
    out_type=(
        jax.ShapeDtypeStruct((B, HID), jnp.float32),
        jax.ShapeDtypeStruct((B, HID), jnp.float32),
    ),
    mesh=_mesh,
    scratch_types=[
        pltpu.VMEM((VOCAB, HIDW), jnp.int32),
        (
            pltpu.VMEM((CHUNK_IDX + 16,), jnp.int32),
            pltpu.VMEM((CHUNK_IDX + 16,), jnp.int32),
        ),
        pltpu.VMEM((2, CHUNK_ROWS, HID), jnp.float32),
        pltpu.SemaphoreType.DMA,
        (pltpu.SemaphoreType.DMA, pltpu.SemaphoreType.DMA),
        (pltpu.SemaphoreType.DMA, pltpu.SemaphoreType.DMA),
    ],
    compiler_params=pltpu.CompilerParams(use_tc_tiling_on_sc=False),
)
def _encode_sc(
    bidx_hbm, cidx_hbm, table_hbm, ctx_hbm, cand_hbm,
    table_v, idx_v, acc_v, tsem, isem, osem,
):
    wid = lax.axis_index("s") * NC + lax.axis_index("c")
    half = wid // (NW // 2)
    base_local = (wid % (NW // 2)) * R_PER_W

    pltpu.async_copy(table_hbm, table_v, tsem)

    def fire_idx(c, b):
        off = pl.multiple_of((base_local + c * CHUNK_ROWS) * L, CHUNK_IDX)
        src = pl.ds(off, CHUNK_IDX)

        dst = idx_v[b].at[pl.ds(0, CHUNK_IDX)]

        @pl.when(half == 0)
        def _():
            pltpu.async_copy(bidx_hbm.at[src], dst, isem[b])

        @pl.when(half == 1)
        def _():
            pltpu.async_copy(cidx_hbm.at[src], dst, isem[b])

    def wait_idx(b):
        pltpu.make_async_copy(
            bidx_hbm.at[pl.ds(0, CHUNK_IDX)],
            idx_v[b].at[pl.ds(0, CHUNK_IDX)],
            isem[b],
        ).wait()

    def fire_out(c, b):
        dst = pl.ds(
            pl.multiple_of(base_local + c * CHUNK_ROWS, CHUNK_ROWS),
            CHUNK_ROWS,
        )

        @pl.when(half == 0)
        def _():
            pltpu.async_copy(acc_v.at[b], ctx_hbm.at[dst], osem[b])

        @pl.when(half == 1)
        def _():
            pltpu.async_copy(acc_v.at[b], cand_hbm.at[dst], osem[b])

    def wait_out(b):
        pltpu.make_async_copy(
            acc_v.at[b], ctx_hbm.at[pl.ds(0, CHUNK_ROWS)], osem[b]
        ).wait()

    def consume(b):
        acc = acc_v.at[b]
        H = CHUNK_ROWS // 4

        def row_body(r, carry):
            accs = []
            for r0 in (r, r + H, r + 2 * H, r + 3 * H):
                b0 = r0 * L
                lo = [jnp.zeros((16,), jnp.float32) for _ in range(NQ)]
                hi = [jnp.zeros((16,), jnp.float32) for _ in range(NQ)]
                ivs = [
                    idx_v[b][pl.ds(b0 + 16 * k, 16)] for k in range(4)
                ]
                for l in range(L):
                    t = ivs[l // 16][l % 16]
                    for s in range(NQ):
                        w = table_v[t, pl.ds(s * 16, 16)]
                        lo[s] = lo[s] + lax.bitcast_convert_type(
                            lax.shift_left(w, 16), jnp.float32
                        )
                        hi[s] = hi[s] + lax.bitcast_convert_type(
                            w, jnp.float32
                        )
                accs.append((lo, hi))
            for r0, (lo, hi) in zip((r, r + H, r + 2 * H, r + 3 * H), accs):
                for s in range(NQ):
                    acc[r0, pl.ds(s * 16, 16)] = lo[s]
                    acc[r0, pl.ds(HIDW + s * 16, 16)] = hi[s]
            return carry

        lax.fori_loop(0, H, row_body, 0)

    fire_idx(0, 0)
    pltpu.make_async_copy(table_hbm, table_v, tsem).wait()

    def ring_body(c, carry):
        for b in range(2):
            cc = c * 2 + b

            @pl.when(cc + 1 < N_CHUNKS)
            def _():
                fire_idx(cc + 1, 1 - b)

            wait_idx(b)

            @pl.when(cc >= 2)
            def _():
                wait_out(b)

            consume(b)
            fire_out(cc, b)
        return carry

    lax.fori_loop(0, N_CHUNKS // 2, ring_body, 0)
    wait_out(0)
    wait_out(1)


def _matmul_body(ctx_ref, cand_ref, o_ref):
    o_ref[...] = lax.dot_general(
        ctx_ref[...].astype(jnp.bfloat16),
        cand_ref[...].astype(jnp.bfloat16),
        (((1,), (1,)), ((), ())),
        preferred_element_type=jnp.float32,
    )


def _similarity(ctx, cand):
    BM = 512
    return pl.pallas_call(
        _matmul_body,
        grid=(B // BM,),
        in_specs=[
            pl.BlockSpec((BM, HID), lambda i: (i, 0)),
            pl.BlockSpec((B, HID), lambda i: (0, 0)),
        ],
        out_specs=pl.BlockSpec((BM, B), lambda i: (i, 0)),
        out_shape=jax.ShapeDtypeStruct((B, B), jnp.float32),
    )(ctx, cand)


def kernel(batch, cand_vecs, W):
    bidx = batch.astype(jnp.int32).reshape(B * L)
    cidx = cand_vecs.astype(jnp.int32).reshape(B * L)
    Wb = (W * (1.0 / L)).astype(jnp.bfloat16)
    Wp = lax.bitcast_convert_type(
        jnp.stack([Wb[:, :HIDW], Wb[:, HIDW:]], axis=-1), jnp.int32
    )
    ctx, cand = _encode_sc(bidx, cidx, Wp)
    return _similarity(ctx, cand)


# speedup vs baseline: 1.0739x; 1.0739x over previous
import functools

import jax
import jax.numpy as jnp
from jax import lax
from jax.experimental import pallas as pl
from jax.experimental.pallas import tpu as pltpu
from jax.experimental.pallas import tpu_sc as plsc

VOCAB = 1000
HID = 128
B = 4096
L = 50

NC = 2
NS = 16
NW = NC * NS
ROWS = 2 * B
R_PER_W = ROWS // NW
CHUNK_ROWS = 16
N_CHUNKS = R_PER_W // CHUNK_ROWS
CHUNK_IDX = CHUNK_ROWS * L
HIDW = HID // 2
NQ = HIDW // 16

_mesh = plsc.VectorSubcoreMesh(core_axis_name="c", subcore_axis_name="s")


@functools.partial(
    pl.kernel,
    out_type=(
        jax.ShapeDtypeStruct((B, HID), jnp.float32),
        jax.ShapeDtypeStruct((B, HID), jnp.float32),
    ),
    mesh=_mesh,
    scratch_types=[
        pltpu.VMEM((VOCAB, HIDW), jnp.int32),
        (
            pltpu.VMEM((CHUNK_IDX + 16,), jnp.int32),
            pltpu.VMEM((CHUNK_IDX + 16,), jnp.int32),
        ),
        pltpu.VMEM((2, CHUNK_ROWS, HID), jnp.float32),
        pltpu.SemaphoreType.DMA,
        (pltpu.SemaphoreType.DMA, pltpu.SemaphoreType.DMA),
        (pltpu.SemaphoreType.DMA, pltpu.SemaphoreType.DMA),
    ],
    compiler_params=pltpu.CompilerParams(use_tc_tiling_on_sc=False),
)
def _encode_sc(
    bidx_hbm, cidx_hbm, table_hbm, ctx_hbm, cand_hbm,
    table_v, idx_v, acc_v, tsem, isem, osem,
):
    wid = lax.axis_index("s") * NC + lax.axis_index("c")
    half = wid // (NW // 2)
    base_local = (wid % (NW // 2)) * R_PER_W

    pltpu.async_copy(table_hbm, table_v, tsem)

    def fire_idx(c, b):
        off = pl.multiple_of((base_local + c * CHUNK_ROWS) * L, CHUNK_IDX)
        src = pl.ds(off, CHUNK_IDX)

        dst = idx_v[b].at[pl.ds(0, CHUNK_IDX)]

        @pl.when(half == 0)
        def _():
            pltpu.async_copy(bidx_hbm.at[src], dst, isem[b])

        @pl.when(half == 1)
        def _():
            pltpu.async_copy(cidx_hbm.at[src], dst, isem[b])

    def wait_idx(b):
        pltpu.make_async_copy(
            bidx_hbm.at[pl.ds(0, CHUNK_IDX)],
            idx_v[b].at[pl.ds(0, CHUNK_IDX)],
            isem[b],
        ).wait()

    def fire_out(c, b):
        dst = pl.ds(
            pl.multiple_of(base_local + c * CHUNK_ROWS, CHUNK_ROWS),
            CHUNK_ROWS,
        )

        @pl.when(half == 0)
        def _():
            pltpu.async_copy(acc_v.at[b], ctx_hbm.at[dst], osem[b])

        @pl.when(half == 1)
        def _():
            pltpu.async_copy(acc_v.at[b], cand_hbm.at[dst], osem[b])

    def wait_out(b):
        pltpu.make_async_copy(
            acc_v.at[b], ctx_hbm.at[pl.ds(0, CHUNK_ROWS)], osem[b]
        ).wait()

    def consume(b):
        acc = acc_v.at[b]
        H = CHUNK_ROWS // 2

        def row_body(r, carry):
            accs = []
            for r0 in (r, r + H):
                b0 = r0 * L
                lo = [jnp.zeros((16,), jnp.float32) for _ in range(NQ)]
                hi = [jnp.zeros((16,), jnp.float32) for _ in range(NQ)]
                ivs = [
                    idx_v[b][pl.ds(b0 + 16 * k, 16)] for k in range(4)
                ]
                for l in range(L):
                    t = ivs[l // 16][l % 16]
                    for s in range(NQ):
                        w = table_v[t, pl.ds(s * 16, 16)]
                        lo[s] = lo[s] + lax.bitcast_convert_type(
                            lax.shift_left(w, 16), jnp.float32
                        )
                        hi[s] = hi[s] + lax.bitcast_convert_type(
                            w, jnp.float32
                        )
                accs.append((lo, hi))
            for r0, (lo, hi) in zip((r, r + H), accs):
                for s in range(NQ):
                    acc[r0, pl.ds(s * 16, 16)] = lo[s]
                    acc[r0, pl.ds(HIDW + s * 16, 16)] = hi[s]
            return carry

        lax.fori_loop(0, H, row_body, 0)

    fire_idx(0, 0)
    pltpu.make_async_copy(table_hbm, table_v, tsem).wait()

    def ring_body(c, carry):
        for b in range(2):
            cc = c * 2 + b

            @pl.when(cc + 1 < N_CHUNKS)
            def _():
                fire_idx(cc + 1, 1 - b)

            wait_idx(b)

            @pl.when(cc >= 2)
            def _():
                wait_out(b)

            consume(b)
            fire_out(cc, b)
        return carry

    lax.fori_loop(0, N_CHUNKS // 2, ring_body, 0)
    wait_out(0)
    wait_out(1)


def _matmul_body(ctx_ref, cand_ref, o_ref):
    o_ref[...] = lax.dot_general(
        ctx_ref[...].astype(jnp.bfloat16),
        cand_ref[...].astype(jnp.bfloat16),
        (((1,), (1,)), ((), ())),
        preferred_element_type=jnp.float32,
    )


def _similarity(ctx, cand):
    BM = 512
    return pl.pallas_call(
        _matmul_body,
        grid=(B // BM,),
        in_specs=[
            pl.BlockSpec((BM, HID), lambda i: (i, 0)),
            pl.BlockSpec((B, HID), lambda i: (0, 0)),
        ],
        out_specs=pl.BlockSpec((BM, B), lambda i: (i, 0)),
        out_shape=jax.ShapeDtypeStruct((B, B), jnp.float32),
    )(ctx, cand)


def kernel(batch, cand_vecs, W):
    bidx = batch.astype(jnp.int32).reshape(B * L)
    cidx = cand_vecs.astype(jnp.int32).reshape(B * L)
    Wb = (W * (1.0 / L)).astype(jnp.bfloat16)
    Wp = lax.bitcast_convert_type(
        jnp.stack([Wb[:, :HIDW], Wb[:, HIDW:]], axis=-1), jnp.int32
    )
    ctx, cand = _encode_sc(bidx, cidx, Wp)
    return _similarity(ctx, cand)
